# baseline (speedup 1.0000x reference)
import functools

import jax
import jax.numpy as jnp
from jax import lax
from jax.experimental import pallas as pl
from jax.experimental.pallas import tpu as pltpu
from jax.experimental.pallas import tpu_sc as plsc

N = 10000
D = 128
E = 320000
G = 64

NC = 2
NS = 16
NW = NC * NS
EPW = E // NW
K = 80
NCH = EPW // K
KB = 40
NBUFB = 7
PFD = 5
NCHB = EPW // KB
UNROLLB = NCHB // NBUFB
EPIL = NCHB - UNROLLB * NBUFB

BN = 1000
NG = N // BN


def _sc_mesh():
    return plsc.VectorSubcoreMesh(
        core_axis_name="c", subcore_axis_name="s", num_cores=NC, num_subcores=NS
    )


NBUF = 5
UNROLL = NCH // NBUF


PFDA = 4


def _deg_body(dst_hbm, zd_hbm, out_hbm, *scr):
    didx = scr[0:NBUF]
    isem = scr[NBUF:2 * NBUF]
    ssem = scr[2 * NBUF:3 * NBUF]
    ones_v = scr[3 * NBUF]
    deg_sh = scr[3 * NBUF + 1]
    c = lax.axis_index("c")
    s = lax.axis_index("s")
    for i in range(K // 16):
        ones_v[pl.ds(i * 16, 16)] = jnp.full((16,), 1.0, jnp.float32)

    @pl.when(s == 0)
    def _init():
        pltpu.sync_copy(zd_hbm.at[c], deg_sh)

    plsc.subcore_barrier()
    base = (c * NS + s) * EPW

    def _stage_idx(ch, b):
        pltpu.async_copy(dst_hbm.at[pl.ds(base + ch * K, K)], didx[b], isem[b])

    def _consume(ch, u):
        pltpu.make_async_copy(dst_hbm.at[pl.ds(base, K)], didx[u],
                              isem[u]).wait()
        pltpu.async_copy(ones_v, deg_sh.at[didx[u]], ssem[u], add=True)

    for u in range(PFDA):
        _stage_idx(u, u)

    def body(cc, carry):
        for u in range(NBUF):
            ch = cc * NBUF + u
            _consume(ch, u)
            b2 = (u + PFDA) % NBUF

            @pl.when(ch + PFDA < NCH)
            def _prefetch():
                @pl.when(ch >= NBUF - PFDA)
                def _drain():
                    pltpu.make_async_copy(ones_v, deg_sh.at[didx[b2]],
                                          ssem[b2]).wait()

                _stage_idx(ch + PFDA, b2)

        return carry

    lax.fori_loop(0, UNROLL, body, 0)
    for ch in range(UNROLL * NBUF, NCH):
        _consume(ch, ch % NBUF)
    for u in range(NBUF):
        pltpu.make_async_copy(ones_v, deg_sh.at[didx[u]], ssem[u]).wait()
    plsc.subcore_barrier()

    @pl.when(s == 0)
    def _out():
        pltpu.sync_copy(deg_sh, out_hbm.at[c])


def _deg_counts(dst32, zd):
    kern = functools.partial(
        pl.kernel,
        mesh=_sc_mesh(),
        out_type=jax.ShapeDtypeStruct((NC, N), jnp.float32),
        scratch_types=(
            [pltpu.VMEM((K,), jnp.int32) for _ in range(NBUF)]
            + [pltpu.SemaphoreType.DMA for _ in range(2 * NBUF)]
            + [pltpu.VMEM((K,), jnp.float32),
               pltpu.VMEM_SHARED((N,), jnp.float32)]
        ),
    )(_deg_body)
    return kern(dst32, zd)


def _prep_body(x_ref, w_ref, d0_ref, d1_ref, hp_ref, dis_ref):
    deg = d0_ref[...] + d1_ref[...] + 1.0
    dis = lax.rsqrt(jnp.maximum(deg, 1e-12))
    h = jnp.dot(x_ref[...], w_ref[...], preferred_element_type=jnp.float32)
    hp_ref[...] = h * dis
    dis_ref[...] = dis


def _prep(x, W, d0, d1):
    return pl.pallas_call(
        _prep_body,
        grid=(NG,),
        in_specs=[
            pl.BlockSpec((BN, D), lambda i: (i, 0)),
            pl.BlockSpec((D, D), lambda i: (0, 0)),
            pl.BlockSpec((BN, 1), lambda i: (i, 0)),
            pl.BlockSpec((BN, 1), lambda i: (i, 0)),
        ],
        out_specs=[
            pl.BlockSpec((BN, D), lambda i: (i, 0)),
            pl.BlockSpec((BN, 1), lambda i: (i, 0)),
        ],
        out_shape=[
            jax.ShapeDtypeStruct((N, D), jnp.float32),
            jax.ShapeDtypeStruct((N, 1), jnp.float32),
        ],
    )(x, W, d0, d1)


NPT = 624
NPT_LAST = N - NPT * (NS - 1)


def _rows_par_copy(s, src_at, dst_at):
    @pl.when(s < NS - 1)
    def _main():
        sl = pl.ds(s * NPT, NPT)
        pltpu.sync_copy(src_at(sl), dst_at(sl))

    @pl.when(s == NS - 1)
    def _last():
        sl = pl.ds(NPT * (NS - 1), NPT_LAST)
        pltpu.sync_copy(src_at(sl), dst_at(sl))


def _edge_body(src_hbm, dst_hbm, hp_hbm, out_hbm, *scr):
    sidx = scr[0]
    didx = scr[1:1 + NBUFB]
    rows = scr[1 + NBUFB:1 + 2 * NBUFB]
    isem = scr[1 + 2 * NBUFB:1 + 3 * NBUFB]
    gsem = scr[1 + 3 * NBUFB:1 + 4 * NBUFB]
    ssem = scr[1 + 4 * NBUFB:1 + 5 * NBUFB]
    acc_sh = scr[1 + 5 * NBUFB]
    c = lax.axis_index("c")
    s = lax.axis_index("s")

    _rows_par_copy(s, lambda sl: hp_hbm.at[sl], lambda sl: acc_sh.at[sl])
    base = (c * NS + s) * EPW
    pltpu.sync_copy(src_hbm.at[pl.ds(base, EPW)], sidx)
    plsc.subcore_barrier()

    def _stage(ch, b):
        pltpu.async_copy(dst_hbm.at[pl.ds(base + ch * KB, KB)], didx[b], isem[b])
        pltpu.async_copy(hp_hbm.at[sidx.at[pl.ds(ch * KB, KB)]], rows[b], gsem[b])

    def _consume(ch, u):
        pltpu.make_async_copy(hp_hbm.at[sidx.at[pl.ds(0, KB)]], rows[u],
                              gsem[u]).wait()
        pltpu.make_async_copy(dst_hbm.at[pl.ds(base, KB)], didx[u],
                              isem[u]).wait()
        pltpu.async_copy(rows[u], acc_sh.at[didx[u]], ssem[u], add=True)

    for u in range(PFD):
        _stage(u, u)

    def body(cc, carry):
        for u in range(NBUFB):
            ch = cc * NBUFB + u
            _consume(ch, u)
            b2 = (u + PFD) % NBUFB

            @pl.when(ch + PFD < NCHB)
            def _prefetch():
                @pl.when(ch >= NBUFB - PFD)
                def _drain():
                    pltpu.make_async_copy(rows[b2], acc_sh.at[didx[b2]],
                                          ssem[b2]).wait()

                _stage(ch + PFD, b2)

        return carry

    lax.fori_loop(0, UNROLLB, body, 0)
    for ch in range(UNROLLB * NBUFB, NCHB):
        _consume(ch, ch % NBUFB)
    for u in range(NBUFB):
        pltpu.make_async_copy(rows[u], acc_sh.at[didx[u]], ssem[u]).wait()
    plsc.subcore_barrier()
    _rows_par_copy(s, lambda sl: acc_sh.at[sl], lambda sl: out_hbm.at[c].at[sl])


def _edge_scatter(src32, dst32, hp):
    kern = functools.partial(
        pl.kernel,
        mesh=_sc_mesh(),
        out_type=jax.ShapeDtypeStruct((NC, N, D), jnp.float32),
        scratch_types=(
            [pltpu.VMEM((EPW,), jnp.int32)]
            + [pltpu.VMEM((KB,), jnp.int32) for _ in range(NBUFB)]
            + [pltpu.VMEM((KB, D), jnp.float32) for _ in range(NBUFB)]
            + [pltpu.SemaphoreType.DMA for _ in range(3 * NBUFB)]
            + [pltpu.VMEM_SHARED((N, D), jnp.float32)]
        ),
    )(_edge_body)
    return kern(src32, dst32, hp)


def _finish_body(a0_ref, a1_ref, hp_ref, dis_ref, b_ref, batch_ref,
                 out_ref, sums, counts):
    i = pl.program_id(0)

    @pl.when(i == 0)
    def _zero():
        sums[...] = jnp.zeros_like(sums)
        counts[...] = jnp.zeros_like(counts)

    rows = (a0_ref[...] + a1_ref[...] - hp_ref[...]) * dis_ref[...] + b_ref[...]
    rows = jnp.maximum(rows, 0.0)
    bvec = batch_ref[0]
    gid = lax.broadcasted_iota(jnp.int32, (G, BN), 0)
    m = jnp.where(bvec == gid, 1.0, 0.0)
    sums[...] += jnp.dot(m, rows, preferred_element_type=jnp.float32)
    counts[...] += jnp.sum(m, axis=1, keepdims=True)

    @pl.when(i == NG - 1)
    def _emit():
        out_ref[...] = sums[...] / jnp.maximum(counts[...], 1.0)


def _finish(a0, a1, hp, dis, b2, batch3):
    return pl.pallas_call(
        _finish_body,
        grid=(NG,),
        in_specs=[
            pl.BlockSpec((BN, D), lambda i: (i, 0)),
            pl.BlockSpec((BN, D), lambda i: (i, 0)),
            pl.BlockSpec((BN, D), lambda i: (i, 0)),
            pl.BlockSpec((BN, 1), lambda i: (i, 0)),
            pl.BlockSpec((1, D), lambda i: (0, 0)),
            pl.BlockSpec((1, 1, BN), lambda i: (i, 0, 0)),
        ],
        out_specs=pl.BlockSpec((G, D), lambda i: (0, 0)),
        out_shape=jax.ShapeDtypeStruct((G, D), jnp.float32),
        scratch_shapes=[
            pltpu.VMEM((G, D), jnp.float32),
            pltpu.VMEM((G, 1), jnp.float32),
        ],
    )(a0, a1, hp, dis, b2, batch3)


def kernel(x, edge_index, batch, W, b):
    src32 = edge_index[0].astype(jnp.int32)
    dst32 = edge_index[1].astype(jnp.int32)
    zd = jnp.zeros((NC, N), jnp.float32)

    dp = _deg_counts(dst32, zd)
    d0 = dp[0].reshape(N, 1)
    d1 = dp[1].reshape(N, 1)

    hp, dis = _prep(x, W, d0, d1)

    acc = _edge_scatter(src32, dst32, hp)

    b2 = b.reshape(1, D)
    batch3 = batch.astype(jnp.int32).reshape(NG, 1, BN)
    return _finish(acc[0], acc[1], hp, dis, b2, batch3)


# speedup vs baseline: 1.0340x; 1.0340x over previous
import functools

import jax
import jax.numpy as jnp
from jax import lax
from jax.experimental import pallas as pl
from jax.experimental.pallas import tpu as pltpu
from jax.experimental.pallas import tpu_sc as plsc

N = 10000
D = 128
E = 320000
G = 64

NC = 2
NS = 16
NW = NC * NS
EPW = E // NW
K = 80
NCH = EPW // K
KB = 40
NBUFB = 7
PFD = 5
NCHB = EPW // KB
UNROLLB = NCHB // NBUFB
EPIL = NCHB - UNROLLB * NBUFB

BN = 1000
NG = N // BN


def _sc_mesh():
    return plsc.VectorSubcoreMesh(
        core_axis_name="c", subcore_axis_name="s", num_cores=NC, num_subcores=NS
    )


NBUF = 5
UNROLL = NCH // NBUF


PFDA = 4


def _deg_body(dst_hbm, zd_hbm, out_hbm, *scr):
    didx = scr[0:NBUF]
    isem = scr[NBUF:2 * NBUF]
    ssem = scr[2 * NBUF:3 * NBUF]
    ones_v = scr[3 * NBUF]
    deg_sh = scr[3 * NBUF + 1]
    c = lax.axis_index("c")
    s = lax.axis_index("s")
    for i in range(K // 16):
        ones_v[pl.ds(i * 16, 16)] = jnp.full((16,), 1.0, jnp.float32)

    @pl.when(s == 0)
    def _init():
        pltpu.sync_copy(zd_hbm.at[c], deg_sh)

    plsc.subcore_barrier()
    base = (c * NS + s) * EPW

    def _stage_idx(ch, b):
        pltpu.async_copy(dst_hbm.at[pl.ds(base + ch * K, K)], didx[b], isem[b])

    def _consume(ch, u):
        pltpu.make_async_copy(dst_hbm.at[pl.ds(base, K)], didx[u],
                              isem[u]).wait()
        pltpu.async_copy(ones_v, deg_sh.at[didx[u]], ssem[u], add=True)

    for u in range(PFDA):
        _stage_idx(u, u)

    def body(cc, carry):
        for u in range(NBUF):
            ch = cc * NBUF + u
            _consume(ch, u)
            b2 = (u + PFDA) % NBUF

            @pl.when(ch + PFDA < NCH)
            def _prefetch():
                @pl.when(ch >= NBUF - PFDA)
                def _drain():
                    pltpu.make_async_copy(ones_v, deg_sh.at[didx[b2]],
                                          ssem[b2]).wait()

                _stage_idx(ch + PFDA, b2)

        return carry

    lax.fori_loop(0, UNROLL, body, 0)
    for ch in range(UNROLL * NBUF, NCH):
        _consume(ch, ch % NBUF)
    for u in range(NBUF):
        pltpu.make_async_copy(ones_v, deg_sh.at[didx[u]], ssem[u]).wait()
    plsc.subcore_barrier()

    @pl.when(s == 0)
    def _out():
        pltpu.sync_copy(deg_sh, out_hbm.at[c])


def _deg_counts(dst32, zd):
    kern = functools.partial(
        pl.kernel,
        mesh=_sc_mesh(),
        out_type=jax.ShapeDtypeStruct((NC, N), jnp.float32),
        scratch_types=(
            [pltpu.VMEM((K,), jnp.int32) for _ in range(NBUF)]
            + [pltpu.SemaphoreType.DMA for _ in range(2 * NBUF)]
            + [pltpu.VMEM((K,), jnp.float32),
               pltpu.VMEM_SHARED((N,), jnp.float32)]
        ),
    )(_deg_body)
    return kern(dst32, zd)


def _prep_body(x_ref, w_ref, d0_ref, d1_ref, hp_ref, dis_ref):
    deg = d0_ref[...] + d1_ref[...] + 1.0
    dis = lax.rsqrt(jnp.maximum(deg, 1e-12))
    h = jnp.dot(x_ref[...], w_ref[...], preferred_element_type=jnp.float32)
    hp_ref[...] = h * dis
    dis_ref[...] = dis


def _prep(x, W, d0, d1):
    return pl.pallas_call(
        _prep_body,
        grid=(NG,),
        in_specs=[
            pl.BlockSpec((BN, D), lambda i: (i, 0)),
            pl.BlockSpec((D, D), lambda i: (0, 0)),
            pl.BlockSpec((BN, 1), lambda i: (i, 0)),
            pl.BlockSpec((BN, 1), lambda i: (i, 0)),
        ],
        out_specs=[
            pl.BlockSpec((BN, D), lambda i: (i, 0)),
            pl.BlockSpec((BN, 1), lambda i: (i, 0)),
        ],
        out_shape=[
            jax.ShapeDtypeStruct((N, D), jnp.float32),
            jax.ShapeDtypeStruct((N, 1), jnp.float32),
        ],
    )(x, W, d0, d1)


NPT = 624
NPT_LAST = N - NPT * (NS - 1)


def _rows_par_copy(s, src_at, dst_at):
    @pl.when(s < NS - 1)
    def _main():
        sl = pl.ds(s * NPT, NPT)
        pltpu.sync_copy(src_at(sl), dst_at(sl))

    @pl.when(s == NS - 1)
    def _last():
        sl = pl.ds(NPT * (NS - 1), NPT_LAST)
        pltpu.sync_copy(src_at(sl), dst_at(sl))


def _edge_body(src_hbm, dst_hbm, hp_hbm, out_hbm, *scr):
    sidx = scr[0]
    didx = scr[1:1 + NBUFB]
    rows = scr[1 + NBUFB:1 + 2 * NBUFB]
    isem = scr[1 + 2 * NBUFB:1 + 3 * NBUFB]
    gsem = scr[1 + 3 * NBUFB:1 + 4 * NBUFB]
    ssem = scr[1 + 4 * NBUFB:1 + 5 * NBUFB]
    acc_sh = scr[1 + 5 * NBUFB]
    c = lax.axis_index("c")
    s = lax.axis_index("s")

    _rows_par_copy(s, lambda sl: hp_hbm.at[sl], lambda sl: acc_sh.at[sl])
    base = (c * NS + s) * EPW
    pltpu.sync_copy(src_hbm.at[pl.ds(base, EPW)], sidx)
    plsc.subcore_barrier()

    def _stage(ch, b):
        pltpu.async_copy(dst_hbm.at[pl.ds(base + ch * KB, KB)], didx[b], isem[b])
        pltpu.async_copy(hp_hbm.at[sidx.at[pl.ds(ch * KB, KB)]], rows[b], gsem[b])

    def _consume(ch, u):
        pltpu.make_async_copy(hp_hbm.at[sidx.at[pl.ds(0, KB)]], rows[u],
                              gsem[u]).wait()
        pltpu.make_async_copy(dst_hbm.at[pl.ds(base, KB)], didx[u],
                              isem[u]).wait()
        pltpu.async_copy(rows[u], acc_sh.at[didx[u]], ssem[u], add=True)

    for u in range(PFD):
        _stage(u, u)

    def body(cc, carry):
        for u in range(NBUFB):
            ch = cc * NBUFB + u
            _consume(ch, u)
            b2 = (u + PFD) % NBUFB

            @pl.when(ch + PFD < NCHB)
            def _prefetch():
                @pl.when(ch >= NBUFB - PFD)
                def _drain():
                    pltpu.make_async_copy(rows[b2], acc_sh.at[didx[b2]],
                                          ssem[b2]).wait()

                _stage(ch + PFD, b2)

        return carry

    lax.fori_loop(0, UNROLLB, body, 0)
    for ch in range(UNROLLB * NBUFB, NCHB):
        _consume(ch, ch % NBUFB)
    for u in range(NBUFB):
        pltpu.make_async_copy(rows[u], acc_sh.at[didx[u]], ssem[u]).wait()
    plsc.subcore_barrier()
    _rows_par_copy(s, lambda sl: acc_sh.at[sl], lambda sl: out_hbm.at[c].at[sl])


def _edge_scatter(src32, dst32, hp):
    kern = functools.partial(
        pl.kernel,
        mesh=_sc_mesh(),
        out_type=jax.ShapeDtypeStruct((NC, N, D), jnp.float32),
        scratch_types=(
            [pltpu.VMEM((EPW,), jnp.int32)]
            + [pltpu.VMEM((KB,), jnp.int32) for _ in range(NBUFB)]
            + [pltpu.VMEM((KB, D), jnp.float32) for _ in range(NBUFB)]
            + [pltpu.SemaphoreType.DMA for _ in range(3 * NBUFB)]
            + [pltpu.VMEM_SHARED((N, D), jnp.float32)]
        ),
    )(_edge_body)
    return kern(src32, dst32, hp)


def _finish_body(a0_ref, a1_ref, hp_ref, dis_ref, b_ref, batch_ref,
                 out_ref, sums, counts):

    i = pl.program_id(0)

    @pl.when(i == 0)
    def _zero():
        sums[...] = jnp.zeros_like(sums)
        counts[...] = jnp.zeros_like(counts)

    acc = a0_ref[0] + a1_ref[0]
    rows = (acc - hp_ref[...]) * dis_ref[...] + b_ref[...]
    rows = jnp.maximum(rows, 0.0)
    bvec = batch_ref[0]
    gid = lax.broadcasted_iota(jnp.int32, (G, BN), 0)
    m = jnp.where(bvec == gid, 1.0, 0.0)
    sums[...] += jnp.dot(m, rows, preferred_element_type=jnp.float32)
    counts[...] += jnp.sum(m, axis=1, keepdims=True)

    @pl.when(i == NG - 1)
    def _emit():
        out_ref[...] = sums[...] / jnp.maximum(counts[...], 1.0)


def _finish(acc, hp, dis, b2, batch3):
    return pl.pallas_call(
        _finish_body,
        grid=(NG,),
        in_specs=[
            pl.BlockSpec((1, BN, D), lambda i: (0, i, 0)),
            pl.BlockSpec((1, BN, D), lambda i: (1, i, 0)),
            pl.BlockSpec((BN, D), lambda i: (i, 0)),
            pl.BlockSpec((BN, 1), lambda i: (i, 0)),
            pl.BlockSpec((1, D), lambda i: (0, 0)),
            pl.BlockSpec((1, 1, BN), lambda i: (i, 0, 0)),
        ],
        out_specs=pl.BlockSpec((G, D), lambda i: (0, 0)),
        out_shape=jax.ShapeDtypeStruct((G, D), jnp.float32),
        scratch_shapes=[
            pltpu.VMEM((G, D), jnp.float32),
            pltpu.VMEM((G, 1), jnp.float32),
        ],
    )(acc, acc, hp, dis, b2, batch3)


def kernel(x, edge_index, batch, W, b):
    dst32 = edge_index[1].astype(jnp.int32)
    src32 = lax.optimization_barrier(edge_index)[0].astype(jnp.int32)
    zd = jnp.zeros((NC, N), jnp.float32)

    dp = _deg_counts(dst32, zd)
    d0 = dp[0].reshape(N, 1)
    d1 = dp[1].reshape(N, 1)

    hp, dis = _prep(x, W, d0, d1)

    acc = _edge_scatter(src32, dst32, hp)

    b2 = b.reshape(1, D)
    batch3 = batch.astype(jnp.int32).reshape(NG, 1, BN)
    return _finish(acc, hp, dis, b2, batch3)


# speedup vs baseline: 1.0772x; 1.0418x over previous
import functools

import jax
import jax.numpy as jnp
from jax import lax
from jax.experimental import pallas as pl
from jax.experimental.pallas import tpu as pltpu
from jax.experimental.pallas import tpu_sc as plsc

N = 10000
D = 128
E = 320000
G = 64

NC = 2
NS = 16
NW = NC * NS
EPW = E // NW
K = 80
NCH = EPW // K
KB = 40
NBUFB = 7
PFD = 5
NCHB = EPW // KB
UNROLLB = NCHB // NBUFB
EPIL = NCHB - UNROLLB * NBUFB

BN = 1000
NG = N // BN


def _sc_mesh():
    return plsc.VectorSubcoreMesh(
        core_axis_name="c", subcore_axis_name="s", num_cores=NC, num_subcores=NS
    )


NBUF = 5
UNROLL = NCH // NBUF


PFDA = 4


def _deg_body(dst_hbm, zd_hbm, out_hbm, *scr):
    didx = scr[0:NBUF]
    isem = scr[NBUF:2 * NBUF]
    ssem = scr[2 * NBUF:3 * NBUF]
    ones_v = scr[3 * NBUF]
    deg_sh = scr[3 * NBUF + 1]
    c = lax.axis_index("c")
    s = lax.axis_index("s")
    for i in range(K // 16):
        ones_v[pl.ds(i * 16, 16)] = jnp.full((16,), 1.0, jnp.float32)

    @pl.when(s == 0)
    def _init():
        pltpu.sync_copy(zd_hbm.at[c], deg_sh)

    plsc.subcore_barrier()
    base = (c * NS + s) * EPW

    def _stage_idx(ch, b):
        pltpu.async_copy(dst_hbm.at[pl.ds(base + ch * K, K)], didx[b], isem[b])

    def _consume(ch, u):
        pltpu.make_async_copy(dst_hbm.at[pl.ds(base, K)], didx[u],
                              isem[u]).wait()
        pltpu.async_copy(ones_v, deg_sh.at[didx[u]], ssem[u], add=True)

    for u in range(PFDA):
        _stage_idx(u, u)

    def body(cc, carry):
        for u in range(NBUF):
            ch = cc * NBUF + u
            _consume(ch, u)
            b2 = (u + PFDA) % NBUF

            @pl.when(ch + PFDA < NCH)
            def _prefetch():
                @pl.when(ch >= NBUF - PFDA)
                def _drain():
                    pltpu.make_async_copy(ones_v, deg_sh.at[didx[b2]],
                                          ssem[b2]).wait()

                _stage_idx(ch + PFDA, b2)

        return carry

    lax.fori_loop(0, UNROLL, body, 0)
    for ch in range(UNROLL * NBUF, NCH):
        _consume(ch, ch % NBUF)
    for u in range(NBUF):
        pltpu.make_async_copy(ones_v, deg_sh.at[didx[u]], ssem[u]).wait()
    plsc.subcore_barrier()

    @pl.when(s == 0)
    def _out():
        pltpu.sync_copy(deg_sh, out_hbm.at[c])


def _deg_counts(dst32, zd):
    kern = functools.partial(
        pl.kernel,
        mesh=_sc_mesh(),
        out_type=jax.ShapeDtypeStruct((NC, N), jnp.float32),
        scratch_types=(
            [pltpu.VMEM((K,), jnp.int32) for _ in range(NBUF)]
            + [pltpu.SemaphoreType.DMA for _ in range(2 * NBUF)]
            + [pltpu.VMEM((K,), jnp.float32),
               pltpu.VMEM_SHARED((N,), jnp.float32)]
        ),
    )(_deg_body)
    return kern(dst32, zd)


def _prep_body(x_ref, w_ref, d0_ref, d1_ref, hp_ref, dis_ref):
    deg = d0_ref[0] + d1_ref[0] + 1.0
    dis_row = lax.rsqrt(jnp.maximum(deg, 1e-12))
    h = jnp.dot(x_ref[...], w_ref[...], preferred_element_type=jnp.float32)
    hp_ref[...] = h * jnp.transpose(dis_row, (1, 0))
    dis_ref[0] = dis_row


def _prep(x, W, d0, d1):
    return pl.pallas_call(
        _prep_body,
        grid=(NG,),
        in_specs=[
            pl.BlockSpec((BN, D), lambda i: (i, 0)),
            pl.BlockSpec((D, D), lambda i: (0, 0)),
            pl.BlockSpec((1, 1, BN), lambda i: (i, 0, 0)),
            pl.BlockSpec((1, 1, BN), lambda i: (i, 0, 0)),
        ],
        out_specs=[
            pl.BlockSpec((BN, D), lambda i: (i, 0)),
            pl.BlockSpec((1, 1, BN), lambda i: (i, 0, 0)),
        ],
        out_shape=[
            jax.ShapeDtypeStruct((N, D), jnp.float32),
            jax.ShapeDtypeStruct((NG, 1, BN), jnp.float32),
        ],
    )(x, W, d0, d1)


NPT = 624
NPT_LAST = N - NPT * (NS - 1)


def _rows_par_copy(s, src_at, dst_at):
    @pl.when(s < NS - 1)
    def _main():
        sl = pl.ds(s * NPT, NPT)
        pltpu.sync_copy(src_at(sl), dst_at(sl))

    @pl.when(s == NS - 1)
    def _last():
        sl = pl.ds(NPT * (NS - 1), NPT_LAST)
        pltpu.sync_copy(src_at(sl), dst_at(sl))


def _edge_body(src_hbm, dst_hbm, hp_hbm, out_hbm, *scr):
    sidx = scr[0]
    didx = scr[1:1 + NBUFB]
    rows = scr[1 + NBUFB:1 + 2 * NBUFB]
    isem = scr[1 + 2 * NBUFB:1 + 3 * NBUFB]
    gsem = scr[1 + 3 * NBUFB:1 + 4 * NBUFB]
    ssem = scr[1 + 4 * NBUFB:1 + 5 * NBUFB]
    acc_sh = scr[1 + 5 * NBUFB]
    c = lax.axis_index("c")
    s = lax.axis_index("s")

    _rows_par_copy(s, lambda sl: hp_hbm.at[sl], lambda sl: acc_sh.at[sl])
    base = (c * NS + s) * EPW
    pltpu.sync_copy(src_hbm.at[pl.ds(base, EPW)], sidx)
    plsc.subcore_barrier()

    def _stage(ch, b):
        pltpu.async_copy(dst_hbm.at[pl.ds(base + ch * KB, KB)], didx[b], isem[b])
        pltpu.async_copy(hp_hbm.at[sidx.at[pl.ds(ch * KB, KB)]], rows[b], gsem[b])

    def _consume(ch, u):
        pltpu.make_async_copy(hp_hbm.at[sidx.at[pl.ds(0, KB)]], rows[u],
                              gsem[u]).wait()
        pltpu.make_async_copy(dst_hbm.at[pl.ds(base, KB)], didx[u],
                              isem[u]).wait()
        pltpu.async_copy(rows[u], acc_sh.at[didx[u]], ssem[u], add=True)

    for u in range(PFD):
        _stage(u, u)

    def body(cc, carry):
        for u in range(NBUFB):
            ch = cc * NBUFB + u
            _consume(ch, u)
            b2 = (u + PFD) % NBUFB

            @pl.when(ch + PFD < NCHB)
            def _prefetch():
                @pl.when(ch >= NBUFB - PFD)
                def _drain():
                    pltpu.make_async_copy(rows[b2], acc_sh.at[didx[b2]],
                                          ssem[b2]).wait()

                _stage(ch + PFD, b2)

        return carry

    lax.fori_loop(0, UNROLLB, body, 0)
    for ch in range(UNROLLB * NBUFB, NCHB):
        _consume(ch, ch % NBUFB)
    for u in range(NBUFB):
        pltpu.make_async_copy(rows[u], acc_sh.at[didx[u]], ssem[u]).wait()
    plsc.subcore_barrier()
    _rows_par_copy(s, lambda sl: acc_sh.at[sl], lambda sl: out_hbm.at[c].at[sl])


def _edge_scatter(src32, dst32, hp):
    kern = functools.partial(
        pl.kernel,
        mesh=_sc_mesh(),
        out_type=jax.ShapeDtypeStruct((NC, N, D), jnp.float32),
        scratch_types=(
            [pltpu.VMEM((EPW,), jnp.int32)]
            + [pltpu.VMEM((KB,), jnp.int32) for _ in range(NBUFB)]
            + [pltpu.VMEM((KB, D), jnp.float32) for _ in range(NBUFB)]
            + [pltpu.SemaphoreType.DMA for _ in range(3 * NBUFB)]
            + [pltpu.VMEM_SHARED((N, D), jnp.float32)]
        ),
    )(_edge_body)
    return kern(src32, dst32, hp)


def _finish_body(a0_ref, a1_ref, hp_ref, dis_ref, b_ref, batch_ref,
                 out_ref, sums, counts):

    i = pl.program_id(0)

    @pl.when(i == 0)
    def _zero():
        sums[...] = jnp.zeros_like(sums)
        counts[...] = jnp.zeros_like(counts)

    acc = a0_ref[0] + a1_ref[0]
    dis_col = jnp.transpose(dis_ref[0], (1, 0))
    rows = (acc - hp_ref[...]) * dis_col + b_ref[...]
    rows = jnp.maximum(rows, 0.0)
    bvec = batch_ref[0]
    gid = lax.broadcasted_iota(jnp.int32, (G, BN), 0)
    m = jnp.where(bvec == gid, 1.0, 0.0)
    sums[...] += jnp.dot(m, rows, preferred_element_type=jnp.float32)
    counts[...] += jnp.sum(m, axis=1, keepdims=True)

    @pl.when(i == NG - 1)
    def _emit():
        out_ref[...] = sums[...] / jnp.maximum(counts[...], 1.0)


def _finish(acc, hp, dis, b2, batch3):
    return pl.pallas_call(
        _finish_body,
        grid=(NG,),
        in_specs=[
            pl.BlockSpec((1, BN, D), lambda i: (0, i, 0)),
            pl.BlockSpec((1, BN, D), lambda i: (1, i, 0)),
            pl.BlockSpec((BN, D), lambda i: (i, 0)),
            pl.BlockSpec((1, 1, BN), lambda i: (i, 0, 0)),
            pl.BlockSpec((1, D), lambda i: (0, 0)),
            pl.BlockSpec((1, 1, BN), lambda i: (i, 0, 0)),
        ],
        out_specs=pl.BlockSpec((G, D), lambda i: (0, 0)),
        out_shape=jax.ShapeDtypeStruct((G, D), jnp.float32),
        scratch_shapes=[
            pltpu.VMEM((G, D), jnp.float32),
            pltpu.VMEM((G, 1), jnp.float32),
        ],
    )(acc, acc, hp, dis, b2, batch3)


def kernel(x, edge_index, batch, W, b):
    dst32 = edge_index[1].astype(jnp.int32)
    src32 = lax.optimization_barrier(edge_index)[0].astype(jnp.int32)
    zd = jnp.zeros((NC, N), jnp.float32)

    dp = _deg_counts(dst32, zd)
    d0 = dp[0].reshape(NG, 1, BN)
    d1 = dp[1].reshape(NG, 1, BN)

    hp, dis = _prep(x, W, d0, d1)

    acc = _edge_scatter(src32, dst32, hp)

    b2 = b.reshape(1, D)
    batch3 = batch.astype(jnp.int32).reshape(NG, 1, BN)
    return _finish(acc, hp, dis, b2, batch3)


# speedup vs baseline: 1.1046x; 1.0254x over previous
import functools

import jax
import jax.numpy as jnp
from jax import lax
from jax.experimental import pallas as pl
from jax.experimental.pallas import tpu as pltpu
from jax.experimental.pallas import tpu_sc as plsc

N = 10000
D = 128
E = 320000
G = 64

NC = 2
NS = 16
NW = NC * NS
EPW = E // NW
K = 80
NCH = EPW // K
KB = 40
NBUFB = 7
PFD = 5
NCHB = EPW // KB
UNROLLB = NCHB // NBUFB
EPIL = NCHB - UNROLLB * NBUFB

BN = 2000
NG = N // BN


def _sc_mesh():
    return plsc.VectorSubcoreMesh(
        core_axis_name="c", subcore_axis_name="s", num_cores=NC, num_subcores=NS
    )


NBUF = 5
UNROLL = NCH // NBUF


PFDA = 4


def _deg_body(dst_hbm, zd_hbm, out_hbm, *scr):
    didx = scr[0:NBUF]
    isem = scr[NBUF:2 * NBUF]
    ssem = scr[2 * NBUF:3 * NBUF]
    ones_v = scr[3 * NBUF]
    deg_sh = scr[3 * NBUF + 1]
    c = lax.axis_index("c")
    s = lax.axis_index("s")
    for i in range(K // 16):
        ones_v[pl.ds(i * 16, 16)] = jnp.full((16,), 1.0, jnp.float32)

    @pl.when(s == 0)
    def _init():
        pltpu.sync_copy(zd_hbm.at[c], deg_sh)

    plsc.subcore_barrier()
    base = (c * NS + s) * EPW

    def _stage_idx(ch, b):
        pltpu.async_copy(dst_hbm.at[pl.ds(base + ch * K, K)], didx[b], isem[b])

    def _consume(ch, u):
        pltpu.make_async_copy(dst_hbm.at[pl.ds(base, K)], didx[u],
                              isem[u]).wait()
        pltpu.async_copy(ones_v, deg_sh.at[didx[u]], ssem[u], add=True)

    for u in range(PFDA):
        _stage_idx(u, u)

    def body(cc, carry):
        for u in range(NBUF):
            ch = cc * NBUF + u
            _consume(ch, u)
            b2 = (u + PFDA) % NBUF

            @pl.when(ch + PFDA < NCH)
            def _prefetch():
                @pl.when(ch >= NBUF - PFDA)
                def _drain():
                    pltpu.make_async_copy(ones_v, deg_sh.at[didx[b2]],
                                          ssem[b2]).wait()

                _stage_idx(ch + PFDA, b2)

        return carry

    lax.fori_loop(0, UNROLL, body, 0)
    for ch in range(UNROLL * NBUF, NCH):
        _consume(ch, ch % NBUF)
    for u in range(NBUF):
        pltpu.make_async_copy(ones_v, deg_sh.at[didx[u]], ssem[u]).wait()
    plsc.subcore_barrier()

    @pl.when(s == 0)
    def _out():
        pltpu.sync_copy(deg_sh, out_hbm.at[c])


def _deg_counts(dst32, zd):
    kern = functools.partial(
        pl.kernel,
        mesh=_sc_mesh(),
        out_type=jax.ShapeDtypeStruct((NC, N), jnp.float32),
        scratch_types=(
            [pltpu.VMEM((K,), jnp.int32) for _ in range(NBUF)]
            + [pltpu.SemaphoreType.DMA for _ in range(2 * NBUF)]
            + [pltpu.VMEM((K,), jnp.float32),
               pltpu.VMEM_SHARED((N,), jnp.float32)]
        ),
    )(_deg_body)
    return kern(dst32, zd)


def _prep_body(x_ref, w_ref, d0_ref, d1_ref, hp_ref, dis_ref):
    deg = d0_ref[0] + d1_ref[0] + 1.0
    dis_row = lax.rsqrt(jnp.maximum(deg, 1e-12))
    h = jnp.dot(x_ref[...], w_ref[...], preferred_element_type=jnp.float32)
    hp_ref[...] = h * jnp.transpose(dis_row, (1, 0))
    dis_ref[0] = dis_row


def _prep(x, W, d0, d1):
    return pl.pallas_call(
        _prep_body,
        grid=(NG,),
        in_specs=[
            pl.BlockSpec((BN, D), lambda i: (i, 0)),
            pl.BlockSpec((D, D), lambda i: (0, 0)),
            pl.BlockSpec((1, 1, BN), lambda i: (i, 0, 0)),
            pl.BlockSpec((1, 1, BN), lambda i: (i, 0, 0)),
        ],
        out_specs=[
            pl.BlockSpec((BN, D), lambda i: (i, 0)),
            pl.BlockSpec((1, 1, BN), lambda i: (i, 0, 0)),
        ],
        out_shape=[
            jax.ShapeDtypeStruct((N, D), jnp.float32),
            jax.ShapeDtypeStruct((NG, 1, BN), jnp.float32),
        ],
    )(x, W, d0, d1)


NPT = 624
NPT_LAST = N - NPT * (NS - 1)


def _rows_par_copy(s, src_at, dst_at):
    @pl.when(s < NS - 1)
    def _main():
        sl = pl.ds(s * NPT, NPT)
        pltpu.sync_copy(src_at(sl), dst_at(sl))

    @pl.when(s == NS - 1)
    def _last():
        sl = pl.ds(NPT * (NS - 1), NPT_LAST)
        pltpu.sync_copy(src_at(sl), dst_at(sl))


def _edge_body(src_hbm, dst_hbm, hp_hbm, out_hbm, *scr):
    sidx = scr[0]
    didx = scr[1:1 + NBUFB]
    rows = scr[1 + NBUFB:1 + 2 * NBUFB]
    isem = scr[1 + 2 * NBUFB:1 + 3 * NBUFB]
    gsem = scr[1 + 3 * NBUFB:1 + 4 * NBUFB]
    ssem = scr[1 + 4 * NBUFB:1 + 5 * NBUFB]
    acc_sh = scr[1 + 5 * NBUFB]
    c = lax.axis_index("c")
    s = lax.axis_index("s")

    _rows_par_copy(s, lambda sl: hp_hbm.at[sl], lambda sl: acc_sh.at[sl])
    base = (c * NS + s) * EPW
    pltpu.sync_copy(src_hbm.at[pl.ds(base, EPW)], sidx)
    plsc.subcore_barrier()

    def _stage(ch, b):
        pltpu.async_copy(dst_hbm.at[pl.ds(base + ch * KB, KB)], didx[b], isem[b])
        pltpu.async_copy(hp_hbm.at[sidx.at[pl.ds(ch * KB, KB)]], rows[b], gsem[b])

    def _consume(ch, u):
        pltpu.make_async_copy(hp_hbm.at[sidx.at[pl.ds(0, KB)]], rows[u],
                              gsem[u]).wait()
        pltpu.make_async_copy(dst_hbm.at[pl.ds(base, KB)], didx[u],
                              isem[u]).wait()
        pltpu.async_copy(rows[u], acc_sh.at[didx[u]], ssem[u], add=True)

    for u in range(PFD):
        _stage(u, u)

    def body(cc, carry):
        for u in range(NBUFB):
            ch = cc * NBUFB + u
            _consume(ch, u)
            b2 = (u + PFD) % NBUFB

            @pl.when(ch + PFD < NCHB)
            def _prefetch():
                @pl.when(ch >= NBUFB - PFD)
                def _drain():
                    pltpu.make_async_copy(rows[b2], acc_sh.at[didx[b2]],
                                          ssem[b2]).wait()

                _stage(ch + PFD, b2)

        return carry

    lax.fori_loop(0, UNROLLB, body, 0)
    for ch in range(UNROLLB * NBUFB, NCHB):
        _consume(ch, ch % NBUFB)
    for u in range(NBUFB):
        pltpu.make_async_copy(rows[u], acc_sh.at[didx[u]], ssem[u]).wait()
    plsc.subcore_barrier()
    _rows_par_copy(s, lambda sl: acc_sh.at[sl], lambda sl: out_hbm.at[c].at[sl])


def _edge_scatter(src32, dst32, hp):
    kern = functools.partial(
        pl.kernel,
        mesh=_sc_mesh(),
        out_type=jax.ShapeDtypeStruct((NC, N, D), jnp.float32),
        scratch_types=(
            [pltpu.VMEM((EPW,), jnp.int32)]
            + [pltpu.VMEM((KB,), jnp.int32) for _ in range(NBUFB)]
            + [pltpu.VMEM((KB, D), jnp.float32) for _ in range(NBUFB)]
            + [pltpu.SemaphoreType.DMA for _ in range(3 * NBUFB)]
            + [pltpu.VMEM_SHARED((N, D), jnp.float32)]
        ),
    )(_edge_body)
    return kern(src32, dst32, hp)


def _finish_body(a0_ref, a1_ref, hp_ref, dis_ref, b_ref, batch_ref,
                 out_ref, sums, counts):

    i = pl.program_id(0)

    @pl.when(i == 0)
    def _zero():
        sums[...] = jnp.zeros_like(sums)
        counts[...] = jnp.zeros_like(counts)

    acc = a0_ref[0] + a1_ref[0]
    dis_col = jnp.transpose(dis_ref[0], (1, 0))
    rows = (acc - hp_ref[...]) * dis_col + b_ref[...]
    rows = jnp.maximum(rows, 0.0)
    bvec = batch_ref[0]
    gid = lax.broadcasted_iota(jnp.int32, (G, BN), 0)
    m = jnp.where(bvec == gid, 1.0, 0.0)
    sums[...] += jnp.dot(m, rows, preferred_element_type=jnp.float32)
    counts[...] += jnp.sum(m, axis=1, keepdims=True)

    @pl.when(i == NG - 1)
    def _emit():
        out_ref[...] = sums[...] / jnp.maximum(counts[...], 1.0)


def _finish(acc, hp, dis, b2, batch3):
    return pl.pallas_call(
        _finish_body,
        grid=(NG,),
        in_specs=[
            pl.BlockSpec((1, BN, D), lambda i: (0, i, 0)),
            pl.BlockSpec((1, BN, D), lambda i: (1, i, 0)),
            pl.BlockSpec((BN, D), lambda i: (i, 0)),
            pl.BlockSpec((1, 1, BN), lambda i: (i, 0, 0)),
            pl.BlockSpec((1, D), lambda i: (0, 0)),
            pl.BlockSpec((1, 1, BN), lambda i: (i, 0, 0)),
        ],
        out_specs=pl.BlockSpec((G, D), lambda i: (0, 0)),
        out_shape=jax.ShapeDtypeStruct((G, D), jnp.float32),
        scratch_shapes=[
            pltpu.VMEM((G, D), jnp.float32),
            pltpu.VMEM((G, 1), jnp.float32),
        ],
    )(acc, acc, hp, dis, b2, batch3)


def kernel(x, edge_index, batch, W, b):
    dst32 = edge_index[1].astype(jnp.int32)
    src32 = lax.optimization_barrier(edge_index)[0].astype(jnp.int32)
    zd = jnp.zeros((NC, N), jnp.float32)

    dp = _deg_counts(dst32, zd)
    d0 = dp[0].reshape(NG, 1, BN)
    d1 = dp[1].reshape(NG, 1, BN)

    hp, dis = _prep(x, W, d0, d1)

    acc = _edge_scatter(src32, dst32, hp)

    b2 = b.reshape(1, D)
    batch3 = batch.astype(jnp.int32).reshape(NG, 1, BN)
    return _finish(acc, hp, dis, b2, batch3)


# speedup vs baseline: 1.1254x; 1.0189x over previous
import functools

import jax
import jax.numpy as jnp
from jax import lax
from jax.experimental import pallas as pl
from jax.experimental.pallas import tpu as pltpu
from jax.experimental.pallas import tpu_sc as plsc

N = 10000
D = 128
E = 320000
G = 64

NC = 2
NS = 16
NW = NC * NS
EPW = E // NW
K = 80
NCH = EPW // K
KB = 40
NBUFB = 7
PFD = 5
NCHB = EPW // KB
UNROLLB = NCHB // NBUFB
EPIL = NCHB - UNROLLB * NBUFB

BN = 5000
NG = N // BN


def _sc_mesh():
    return plsc.VectorSubcoreMesh(
        core_axis_name="c", subcore_axis_name="s", num_cores=NC, num_subcores=NS
    )


NBUF = 5
UNROLL = NCH // NBUF


PFDA = 4


def _deg_body(dst_hbm, zd_hbm, out_hbm, *scr):
    didx = scr[0:NBUF]
    isem = scr[NBUF:2 * NBUF]
    ssem = scr[2 * NBUF:3 * NBUF]
    ones_v = scr[3 * NBUF]
    deg_sh = scr[3 * NBUF + 1]
    c = lax.axis_index("c")
    s = lax.axis_index("s")
    for i in range(K // 16):
        ones_v[pl.ds(i * 16, 16)] = jnp.full((16,), 1.0, jnp.float32)

    @pl.when(s == 0)
    def _init():
        pltpu.sync_copy(zd_hbm.at[c], deg_sh)

    plsc.subcore_barrier()
    base = (c * NS + s) * EPW

    def _stage_idx(ch, b):
        pltpu.async_copy(dst_hbm.at[pl.ds(base + ch * K, K)], didx[b], isem[b])

    def _consume(ch, u):
        pltpu.make_async_copy(dst_hbm.at[pl.ds(base, K)], didx[u],
                              isem[u]).wait()
        pltpu.async_copy(ones_v, deg_sh.at[didx[u]], ssem[u], add=True)

    for u in range(PFDA):
        _stage_idx(u, u)

    def body(cc, carry):
        for u in range(NBUF):
            ch = cc * NBUF + u
            _consume(ch, u)
            b2 = (u + PFDA) % NBUF

            @pl.when(ch + PFDA < NCH)
            def _prefetch():
                @pl.when(ch >= NBUF - PFDA)
                def _drain():
                    pltpu.make_async_copy(ones_v, deg_sh.at[didx[b2]],
                                          ssem[b2]).wait()

                _stage_idx(ch + PFDA, b2)

        return carry

    lax.fori_loop(0, UNROLL, body, 0)
    for ch in range(UNROLL * NBUF, NCH):
        _consume(ch, ch % NBUF)
    for u in range(NBUF):
        pltpu.make_async_copy(ones_v, deg_sh.at[didx[u]], ssem[u]).wait()
    plsc.subcore_barrier()

    @pl.when(s == 0)
    def _out():
        pltpu.sync_copy(deg_sh, out_hbm.at[c])


def _deg_counts(dst32, zd):
    kern = functools.partial(
        pl.kernel,
        mesh=_sc_mesh(),
        out_type=jax.ShapeDtypeStruct((NC, N), jnp.float32),
        scratch_types=(
            [pltpu.VMEM((K,), jnp.int32) for _ in range(NBUF)]
            + [pltpu.SemaphoreType.DMA for _ in range(2 * NBUF)]
            + [pltpu.VMEM((K,), jnp.float32),
               pltpu.VMEM_SHARED((N,), jnp.float32)]
        ),
    )(_deg_body)
    return kern(dst32, zd)


def _prep_body(x_ref, w_ref, d0_ref, d1_ref, hp_ref, dis_ref):
    deg = d0_ref[0] + d1_ref[0] + 1.0
    dis_row = lax.rsqrt(jnp.maximum(deg, 1e-12))
    h = jnp.dot(x_ref[...], w_ref[...], preferred_element_type=jnp.float32)
    hp_ref[...] = h * jnp.transpose(dis_row, (1, 0))
    dis_ref[0] = dis_row


def _prep(x, W, d0, d1):
    return pl.pallas_call(
        _prep_body,
        grid=(NG,),
        in_specs=[
            pl.BlockSpec((BN, D), lambda i: (i, 0)),
            pl.BlockSpec((D, D), lambda i: (0, 0)),
            pl.BlockSpec((1, 1, BN), lambda i: (i, 0, 0)),
            pl.BlockSpec((1, 1, BN), lambda i: (i, 0, 0)),
        ],
        out_specs=[
            pl.BlockSpec((BN, D), lambda i: (i, 0)),
            pl.BlockSpec((1, 1, BN), lambda i: (i, 0, 0)),
        ],
        out_shape=[
            jax.ShapeDtypeStruct((N, D), jnp.float32),
            jax.ShapeDtypeStruct((NG, 1, BN), jnp.float32),
        ],
    )(x, W, d0, d1)


NPT = 624
NPT_LAST = N - NPT * (NS - 1)


def _rows_par_copy(s, src_at, dst_at):
    @pl.when(s < NS - 1)
    def _main():
        sl = pl.ds(s * NPT, NPT)
        pltpu.sync_copy(src_at(sl), dst_at(sl))

    @pl.when(s == NS - 1)
    def _last():
        sl = pl.ds(NPT * (NS - 1), NPT_LAST)
        pltpu.sync_copy(src_at(sl), dst_at(sl))


def _edge_body(src_hbm, dst_hbm, hp_hbm, out_hbm, *scr):
    sidx = scr[0]
    didx = scr[1:1 + NBUFB]
    rows = scr[1 + NBUFB:1 + 2 * NBUFB]
    isem = scr[1 + 2 * NBUFB:1 + 3 * NBUFB]
    gsem = scr[1 + 3 * NBUFB:1 + 4 * NBUFB]
    ssem = scr[1 + 4 * NBUFB:1 + 5 * NBUFB]
    acc_sh = scr[1 + 5 * NBUFB]
    c = lax.axis_index("c")
    s = lax.axis_index("s")

    _rows_par_copy(s, lambda sl: hp_hbm.at[sl], lambda sl: acc_sh.at[sl])
    base = (c * NS + s) * EPW
    pltpu.sync_copy(src_hbm.at[pl.ds(base, EPW)], sidx)
    plsc.subcore_barrier()

    def _stage(ch, b):
        pltpu.async_copy(dst_hbm.at[pl.ds(base + ch * KB, KB)], didx[b], isem[b])
        pltpu.async_copy(hp_hbm.at[sidx.at[pl.ds(ch * KB, KB)]], rows[b], gsem[b])

    def _consume(ch, u):
        pltpu.make_async_copy(hp_hbm.at[sidx.at[pl.ds(0, KB)]], rows[u],
                              gsem[u]).wait()
        pltpu.make_async_copy(dst_hbm.at[pl.ds(base, KB)], didx[u],
                              isem[u]).wait()
        pltpu.async_copy(rows[u], acc_sh.at[didx[u]], ssem[u], add=True)

    for u in range(PFD):
        _stage(u, u)

    def body(cc, carry):
        for u in range(NBUFB):
            ch = cc * NBUFB + u
            _consume(ch, u)
            b2 = (u + PFD) % NBUFB

            @pl.when(ch + PFD < NCHB)
            def _prefetch():
                @pl.when(ch >= NBUFB - PFD)
                def _drain():
                    pltpu.make_async_copy(rows[b2], acc_sh.at[didx[b2]],
                                          ssem[b2]).wait()

                _stage(ch + PFD, b2)

        return carry

    lax.fori_loop(0, UNROLLB, body, 0)
    for ch in range(UNROLLB * NBUFB, NCHB):
        _consume(ch, ch % NBUFB)
    for u in range(NBUFB):
        pltpu.make_async_copy(rows[u], acc_sh.at[didx[u]], ssem[u]).wait()
    plsc.subcore_barrier()
    _rows_par_copy(s, lambda sl: acc_sh.at[sl], lambda sl: out_hbm.at[c].at[sl])


def _edge_scatter(src32, dst32, hp):
    kern = functools.partial(
        pl.kernel,
        mesh=_sc_mesh(),
        out_type=jax.ShapeDtypeStruct((NC, N, D), jnp.float32),
        scratch_types=(
            [pltpu.VMEM((EPW,), jnp.int32)]
            + [pltpu.VMEM((KB,), jnp.int32) for _ in range(NBUFB)]
            + [pltpu.VMEM((KB, D), jnp.float32) for _ in range(NBUFB)]
            + [pltpu.SemaphoreType.DMA for _ in range(3 * NBUFB)]
            + [pltpu.VMEM_SHARED((N, D), jnp.float32)]
        ),
    )(_edge_body)
    return kern(src32, dst32, hp)


def _finish_body(a0_ref, a1_ref, hp_ref, dis_ref, b_ref, batch_ref,
                 out_ref, sums, counts):

    i = pl.program_id(0)

    @pl.when(i == 0)
    def _zero():
        sums[...] = jnp.zeros_like(sums)
        counts[...] = jnp.zeros_like(counts)

    acc = a0_ref[0] + a1_ref[0]
    dis_col = jnp.transpose(dis_ref[0], (1, 0))
    rows = (acc - hp_ref[...]) * dis_col + b_ref[...]
    rows = jnp.maximum(rows, 0.0)
    bvec = batch_ref[0]
    gid = lax.broadcasted_iota(jnp.int32, (G, BN), 0)
    m = jnp.where(bvec == gid, 1.0, 0.0)
    sums[...] += jnp.dot(m, rows, preferred_element_type=jnp.float32)
    counts[...] += jnp.sum(m, axis=1, keepdims=True)

    @pl.when(i == NG - 1)
    def _emit():
        out_ref[...] = sums[...] / jnp.maximum(counts[...], 1.0)


def _finish(acc, hp, dis, b2, batch3):
    return pl.pallas_call(
        _finish_body,
        grid=(NG,),
        in_specs=[
            pl.BlockSpec((1, BN, D), lambda i: (0, i, 0)),
            pl.BlockSpec((1, BN, D), lambda i: (1, i, 0)),
            pl.BlockSpec((BN, D), lambda i: (i, 0)),
            pl.BlockSpec((1, 1, BN), lambda i: (i, 0, 0)),
            pl.BlockSpec((1, D), lambda i: (0, 0)),
            pl.BlockSpec((1, 1, BN), lambda i: (i, 0, 0)),
        ],
        out_specs=pl.BlockSpec((G, D), lambda i: (0, 0)),
        out_shape=jax.ShapeDtypeStruct((G, D), jnp.float32),
        scratch_shapes=[
            pltpu.VMEM((G, D), jnp.float32),
            pltpu.VMEM((G, 1), jnp.float32),
        ],
    )(acc, acc, hp, dis, b2, batch3)


def kernel(x, edge_index, batch, W, b):
    dst32 = edge_index[1].astype(jnp.int32)
    src32 = lax.optimization_barrier(edge_index)[0].astype(jnp.int32)
    zd = jnp.zeros((NC, N), jnp.float32)

    dp = _deg_counts(dst32, zd)
    d0 = dp[0].reshape(NG, 1, BN)
    d1 = dp[1].reshape(NG, 1, BN)

    hp, dis = _prep(x, W, d0, d1)

    acc = _edge_scatter(src32, dst32, hp)

    b2 = b.reshape(1, D)
    batch3 = batch.astype(jnp.int32).reshape(NG, 1, BN)
    return _finish(acc, hp, dis, b2, batch3)
